# Initial kernel scaffold; baseline (speedup 1.0000x reference)
import jax
import jax.numpy as jnp
from jax.experimental import pallas as pl


def kernel(x, rois, roi_indices, w1, b1, w2, b2, w_loc, b_loc, w_score, b_score):
    raise NotImplementedError("write your pallas kernel here")


# speedup vs baseline: 18.9750x; 18.9750x over previous
import functools

import jax
import jax.numpy as jnp
from jax.experimental import pallas as pl
from jax.experimental.pallas import tpu as pltpu

_OUT = 7
_SCALE = 1.0 / 16.0
_NEG = float("-inf")


def _pool_kernel(ws_r, we_r, hs_r, he_r, ws9_r, hs9_r, bidx_r,
                 x_r, out_r, cm_r, *, rb, H, C):
    i = pl.program_id(0)
    for j in range(rb):
        r = i * rb + j
        b = bidx_r[r]
        for pw in range(_OUT):
            st = ws9_r[r * _OUT + pw]
            lo = ws_r[r * _OUT + pw]
            hi = we_r[r * _OUT + pw]
            sl = x_r[b, pl.ds(st, 9)]
            wi = jax.lax.broadcasted_iota(jnp.int32, (9, 1, 1), 0) + st
            m = (wi >= lo) & (wi < hi)
            cm_r[pw, 0:H, :] = jnp.max(jnp.where(m, sl, _NEG), axis=0)
        for ph in range(_OUT):
            st = hs9_r[r * _OUT + ph]
            lo = hs_r[r * _OUT + ph]
            hi = he_r[r * _OUT + ph]
            st8 = pl.multiple_of((st >> 3) << 3, 8)
            ch = cm_r[:, pl.ds(st8, 16), :]
            hi_iota = jax.lax.broadcasted_iota(jnp.int32, (1, 16, 1), 1) + st8
            m = (hi_iota >= lo) & (hi_iota < hi)
            red = jnp.max(jnp.where(m, ch, _NEG), axis=1)
            res = jnp.where(red == _NEG, jnp.float32(0.0), red)
            out_r[j, ph * _OUT:(ph + 1) * _OUT, :] = res[:_OUT]


def _mlp_kernel(flat_r, w1_r, b1_r, w2_r, b2_r, wl_r, bl_r, wsc_r, bsc_r,
                locs_r, scores_r, acc_r, *, kg):
    k = pl.program_id(1)

    @pl.when(k == 0)
    def _():
        acc_r[...] = jnp.zeros_like(acc_r)

    dn = (((1,), (1,)), ((), ()))
    acc_r[...] += jax.lax.dot_general(
        flat_r[...], w1_r[...], dn, preferred_element_type=jnp.float32)

    @pl.when(k == kg - 1)
    def _():
        h1 = jnp.maximum(acc_r[...] + b1_r[...], 0.0)
        h2 = jnp.maximum(
            jax.lax.dot_general(h1, w2_r[...], dn,
                                preferred_element_type=jnp.float32)
            + b2_r[...], 0.0)
        locs_r[...] = jax.lax.dot_general(
            h2, wl_r[...], dn, preferred_element_type=jnp.float32) + bl_r[...]
        scores_r[...] = jax.lax.dot_general(
            h2, wsc_r[...], dn,
            preferred_element_type=jnp.float32) + bsc_r[...]


def kernel(x, rois, roi_indices, w1, b1, w2, b2, w_loc, b_loc, w_score, b_score):
    B, C, H, W = x.shape
    R = rois.shape[0]
    fc = w1.shape[0]
    nl = w_loc.shape[0]
    ns = w_score.shape[0]

    xyxy = rois[:, jnp.array([1, 0, 3, 2])]
    s = jnp.round(xyxy * _SCALE).astype(jnp.int32)
    x1, y1, x2, y2 = s[:, 0], s[:, 1], s[:, 2], s[:, 3]
    roi_w = jnp.maximum(x2 - x1, 1).astype(jnp.float32)
    roi_h = jnp.maximum(y2 - y1, 1).astype(jnp.float32)
    bw = roi_w / _OUT
    bh = roi_h / _OUT
    p = jnp.arange(_OUT, dtype=jnp.float32)
    hs = jnp.clip(jnp.floor(p[None, :] * bh[:, None]).astype(jnp.int32)
                  + y1[:, None], 0, H)
    he = jnp.clip(jnp.ceil((p[None, :] + 1.0) * bh[:, None]).astype(jnp.int32)
                  + y1[:, None], 0, H)
    ws = jnp.clip(jnp.floor(p[None, :] * bw[:, None]).astype(jnp.int32)
                  + x1[:, None], 0, W)
    we = jnp.clip(jnp.ceil((p[None, :] + 1.0) * bw[:, None]).astype(jnp.int32)
                  + x1[:, None], 0, W)
    ws9 = jnp.minimum(ws, W - 9)
    hs9 = jnp.minimum(hs, H - 9)
    flat1 = lambda a: a.reshape(-1)

    x_t = jnp.transpose(x, (0, 3, 2, 1))

    rb = 8
    smem = pl.BlockSpec(memory_space=pltpu.SMEM)
    pool = pl.pallas_call(
        functools.partial(_pool_kernel, rb=rb, H=H, C=C),
        grid=(R // rb,),
        in_specs=[smem] * 7 + [
            pl.BlockSpec((B, W, H, C), lambda i: (0, 0, 0, 0)),
        ],
        out_specs=pl.BlockSpec((rb, _OUT * _OUT, C), lambda i: (i, 0, 0)),
        out_shape=jax.ShapeDtypeStruct((R, _OUT * _OUT, C), jnp.float32),
        scratch_shapes=[pltpu.VMEM((8, 64, C), jnp.float32)],
        compiler_params=pltpu.CompilerParams(
            dimension_semantics=("parallel",)),
        name="roi_max_pool",
    )(flat1(ws), flat1(we), flat1(hs), flat1(he), flat1(ws9), flat1(hs9),
      roi_indices, x_t)

    flat = jnp.transpose(pool, (0, 2, 1)).reshape(R, C * _OUT * _OUT)

    rg = 2
    kg = 7
    kb = (C * _OUT * _OUT) // kg
    mr = R // rg
    locs, scores = pl.pallas_call(
        functools.partial(_mlp_kernel, kg=kg),
        grid=(rg, kg),
        in_specs=[
            pl.BlockSpec((mr, kb), lambda i, k: (i, k)),
            pl.BlockSpec((fc, kb), lambda i, k: (0, k)),
            pl.BlockSpec((1, fc), lambda i, k: (0, 0)),
            pl.BlockSpec((fc, fc), lambda i, k: (0, 0)),
            pl.BlockSpec((1, fc), lambda i, k: (0, 0)),
            pl.BlockSpec((nl, fc), lambda i, k: (0, 0)),
            pl.BlockSpec((1, nl), lambda i, k: (0, 0)),
            pl.BlockSpec((ns, fc), lambda i, k: (0, 0)),
            pl.BlockSpec((1, ns), lambda i, k: (0, 0)),
        ],
        out_specs=[
            pl.BlockSpec((mr, nl), lambda i, k: (i, 0)),
            pl.BlockSpec((mr, ns), lambda i, k: (i, 0)),
        ],
        out_shape=[
            jax.ShapeDtypeStruct((R, nl), jnp.float32),
            jax.ShapeDtypeStruct((R, ns), jnp.float32),
        ],
        scratch_shapes=[pltpu.VMEM((mr, fc), jnp.float32)],
        compiler_params=pltpu.CompilerParams(
            dimension_semantics=("parallel", "arbitrary")),
        name="roi_mlp_head",
    )(flat, w1, b1.reshape(1, fc), w2, b2.reshape(1, fc),
      w_loc, bl_2d(b_loc, nl), w_score, bl_2d(b_score, ns))

    return (locs, scores)


def bl_2d(b, n):
    return b.reshape(1, n)


# speedup vs baseline: 25.6807x; 1.3534x over previous
import functools

import jax
import jax.numpy as jnp
from jax.experimental import pallas as pl
from jax.experimental.pallas import tpu as pltpu

_OUT = 7
_SCALE = 1.0 / 16.0
_NEG = float("-inf")


def _pool_kernel(ws_r, we_r, hs_r, he_r, ws9_r, hs9_r, bidx_r,
                 x_r, out_r, cm_r, *, rb, H, C):
    i = pl.program_id(0)
    for j in range(rb):
        r = i * rb + j
        b = bidx_r[r]
        for pw in range(_OUT):
            st = ws9_r[r * _OUT + pw]
            lo = ws_r[r * _OUT + pw]
            hi = we_r[r * _OUT + pw]
            sl = x_r[b, pl.ds(st, 9)]
            wi = jax.lax.broadcasted_iota(jnp.int32, (9, 1, 1), 0) + st
            m = (wi >= lo) & (wi < hi)
            cm_r[pw, 0:H, :] = jnp.max(jnp.where(m, sl, _NEG), axis=0)
        for ph in range(_OUT):
            st = hs9_r[r * _OUT + ph]
            lo = hs_r[r * _OUT + ph]
            hi = he_r[r * _OUT + ph]
            st8 = pl.multiple_of((st >> 3) << 3, 8)
            ch = cm_r[:, pl.ds(st8, 16), :]
            hi_iota = jax.lax.broadcasted_iota(jnp.int32, (1, 16, 1), 1) + st8
            m = (hi_iota >= lo) & (hi_iota < hi)
            red = jnp.max(jnp.where(m, ch, _NEG), axis=1)
            res = jnp.where(red == _NEG, jnp.bfloat16(0.0), red)
            out_r[j, ph * _OUT:(ph + 1) * _OUT, :] = res[:_OUT]


def _mlp_kernel(flat_r, w1_r, b1_r, w2_r, b2_r, wl_r, bl_r, wsc_r, bsc_r,
                locs_r, scores_r, acc_r, *, kg):
    k = pl.program_id(1)

    @pl.when(k == 0)
    def _():
        acc_r[...] = jnp.zeros_like(acc_r)

    bf = jnp.bfloat16
    dn = (((1,), (1,)), ((), ()))
    acc_r[...] += jax.lax.dot_general(
        flat_r[...], w1_r[...].astype(bf), dn,
        preferred_element_type=jnp.float32)

    @pl.when(k == kg - 1)
    def _():
        h1 = jnp.maximum(acc_r[...] + b1_r[...], 0.0).astype(bf)
        h2 = jnp.maximum(
            jax.lax.dot_general(h1, w2_r[...].astype(bf), dn,
                                preferred_element_type=jnp.float32)
            + b2_r[...], 0.0).astype(bf)
        locs_r[...] = jax.lax.dot_general(
            h2, wl_r[...].astype(bf), dn,
            preferred_element_type=jnp.float32) + bl_r[...]
        scores_r[...] = jax.lax.dot_general(
            h2, wsc_r[...].astype(bf), dn,
            preferred_element_type=jnp.float32) + bsc_r[...]


def kernel(x, rois, roi_indices, w1, b1, w2, b2, w_loc, b_loc, w_score, b_score):
    B, C, H, W = x.shape
    R = rois.shape[0]
    fc = w1.shape[0]
    nl = w_loc.shape[0]
    ns = w_score.shape[0]

    xyxy = rois[:, jnp.array([1, 0, 3, 2])]
    s = jnp.round(xyxy * _SCALE).astype(jnp.int32)
    x1, y1, x2, y2 = s[:, 0], s[:, 1], s[:, 2], s[:, 3]
    roi_w = jnp.maximum(x2 - x1, 1).astype(jnp.float32)
    roi_h = jnp.maximum(y2 - y1, 1).astype(jnp.float32)
    bw = roi_w / _OUT
    bh = roi_h / _OUT
    p = jnp.arange(_OUT, dtype=jnp.float32)
    hs = jnp.clip(jnp.floor(p[None, :] * bh[:, None]).astype(jnp.int32)
                  + y1[:, None], 0, H)
    he = jnp.clip(jnp.ceil((p[None, :] + 1.0) * bh[:, None]).astype(jnp.int32)
                  + y1[:, None], 0, H)
    ws = jnp.clip(jnp.floor(p[None, :] * bw[:, None]).astype(jnp.int32)
                  + x1[:, None], 0, W)
    we = jnp.clip(jnp.ceil((p[None, :] + 1.0) * bw[:, None]).astype(jnp.int32)
                  + x1[:, None], 0, W)
    ws9 = jnp.minimum(ws, W - 9)
    hs9 = jnp.minimum(hs, H - 9)
    flat1 = lambda a: a.reshape(-1)

    x_t = jnp.transpose(x, (0, 3, 2, 1)).astype(jnp.bfloat16)

    rb = 8
    smem = pl.BlockSpec(memory_space=pltpu.SMEM)
    pool = pl.pallas_call(
        functools.partial(_pool_kernel, rb=rb, H=H, C=C),
        grid=(R // rb,),
        in_specs=[smem] * 7 + [
            pl.BlockSpec((B, W, H, C), lambda i: (0, 0, 0, 0)),
        ],
        out_specs=pl.BlockSpec((rb, _OUT * _OUT, C), lambda i: (i, 0, 0)),
        out_shape=jax.ShapeDtypeStruct((R, _OUT * _OUT, C), jnp.bfloat16),
        scratch_shapes=[pltpu.VMEM((8, 64, C), jnp.bfloat16)],
        compiler_params=pltpu.CompilerParams(
            dimension_semantics=("parallel",)),
        name="roi_max_pool",
    )(flat1(ws), flat1(we), flat1(hs), flat1(he), flat1(ws9), flat1(hs9),
      roi_indices, x_t)

    flat = jnp.transpose(pool, (0, 2, 1)).reshape(R, C * _OUT * _OUT)

    rg = 2
    kg = 7
    kb = (C * _OUT * _OUT) // kg
    mr = R // rg
    locs, scores = pl.pallas_call(
        functools.partial(_mlp_kernel, kg=kg),
        grid=(rg, kg),
        in_specs=[
            pl.BlockSpec((mr, kb), lambda i, k: (i, k)),
            pl.BlockSpec((fc, kb), lambda i, k: (0, k)),
            pl.BlockSpec((1, fc), lambda i, k: (0, 0)),
            pl.BlockSpec((fc, fc), lambda i, k: (0, 0)),
            pl.BlockSpec((1, fc), lambda i, k: (0, 0)),
            pl.BlockSpec((nl, fc), lambda i, k: (0, 0)),
            pl.BlockSpec((1, nl), lambda i, k: (0, 0)),
            pl.BlockSpec((ns, fc), lambda i, k: (0, 0)),
            pl.BlockSpec((1, ns), lambda i, k: (0, 0)),
        ],
        out_specs=[
            pl.BlockSpec((mr, nl), lambda i, k: (i, 0)),
            pl.BlockSpec((mr, ns), lambda i, k: (i, 0)),
        ],
        out_shape=[
            jax.ShapeDtypeStruct((R, nl), jnp.float32),
            jax.ShapeDtypeStruct((R, ns), jnp.float32),
        ],
        scratch_shapes=[pltpu.VMEM((mr, fc), jnp.float32)],
        compiler_params=pltpu.CompilerParams(
            dimension_semantics=("parallel", "arbitrary")),
        name="roi_mlp_head",
    )(flat, w1, b1.reshape(1, fc), w2, b2.reshape(1, fc),
      w_loc, bl_2d(b_loc, nl), w_score, bl_2d(b_score, ns))

    return (locs, scores)


def bl_2d(b, n):
    return b.reshape(1, n)
